# baseline (speedup 1.0000x reference)
import functools

import jax
import jax.numpy as jnp
from jax import lax
from jax.experimental import pallas as pl
from jax.experimental.pallas import tpu as pltpu
from jax.experimental.pallas import tpu_sc as plsc

NC = 2
NS = 16
L = 16

N_SIDE = 2048

MAT_N = (5000, 2500, 1250, 4000, 4500, 5000)
MAT_C = (256, 128, 128, 128, 256, 256)
MAT_PADDED = tuple(32 * c for c in MAT_C)
MAT_OFF = (0, 8192, 12288, 16384, 20480, 28672)

GT_N = 20000
GT_CH = 4096
GT_PAD = 20480
GT_CHUNKS = GT_PAD // GT_CH

Q_N = (3000, 1500, 750)
Q_PAD = (3072, 1536, 768)
Q_OFF = (0, 3072, 4608)
Q_TOTAL = 5376
Q_ROWS = (24, 12, 6)
Q_ROW0 = (0, 24, 36)

SMAP_BODY = 512 * 2048
SMAP_WD = SMAP_BODY
SMAP_RD = SMAP_BODY + 512
SMAP_LEN = SMAP_BODY + 1024

ZBUF = 4096


def _sc_body(gt_flat, mat_r, mat_c, gtp_r, gtp_c, gtp_ov, q_r, q_c,
             partials_out,
             sg_r, sg_c, sg_ov, sq_r, sq_c, idx_a, idx_q, gq,
             zeros_v, ones_v, stage_r, stage_c, idx_m, vals_m, partial_v,
             smap, sem_big, sem_z, sem_m, sem_s, sem_g):
    cid = lax.axis_index("c")
    sid = lax.axis_index("s")
    tid = cid * NS + sid
    wid = sid * NC + cid
    iota16 = lax.iota(jnp.int32, L)

    big_descs = [
        pltpu.async_copy(q_r, sq_r, sem_big),
        pltpu.async_copy(q_c, sq_c, sem_big),
    ]

    def _init_zeros(i, c):
        zeros_v[pl.ds(i * L, L)] = jnp.zeros((L,), jnp.float32)
        return c
    lax.fori_loop(0, ZBUF // L, _init_zeros, 0)
    for t in range(8):
        ones_v[pl.ds(t * L, L)] = jnp.full((L,), 1.0, jnp.float32)
    partial_v[...] = jnp.zeros((L,), jnp.float32)

    zbase = sid * 65536
    zdescs = [
        pltpu.async_copy(zeros_v, smap.at[pl.ds(zbase + t * ZBUF, ZBUF)], sem_z)
        for t in range(8)
    ]
    zdescs.append(pltpu.async_copy(zeros_v.at[pl.ds(0, 32)],
                                   smap.at[pl.ds(SMAP_RD + sid * 32, 32)],
                                   sem_z))

    MTOT = sum(MAT_C)
    stg = []
    voff = 0
    for g in range(6):
        C = MAT_C[g]
        base = wid * C
        stg.append(pltpu.async_copy(mat_r.at[pl.ds(MAT_OFF[g] + base, C)],
                                    stage_r.at[pl.ds(voff, C)], sem_m))
        stg.append(pltpu.async_copy(mat_c.at[pl.ds(MAT_OFF[g] + base, C)],
                                    stage_c.at[pl.ds(voff, C)], sem_m))
        voff += C
    for d in stg:
        d.wait()
    for j in range(MTOT // L):
        rv = stage_r[pl.ds(j * L, L)]
        cv = stage_c[pl.ds(j * L, L)]
        idx_m[j // 8, pl.ds((j % 8) * L, L)] = rv * N_SIDE + cv
    mdescs = [
        pltpu.async_copy(gt_flat.at[idx_m.at[t]],
                         vals_m.at[pl.ds(t * 128, 128)], sem_m)
        for t in range(MTOT // 128)
    ]
    for d in mdescs:
        d.wait()
    voff = 0
    for g in range(6):
        C = MAT_C[g]
        base = wid * C
        acc = jnp.zeros((L,), jnp.float32)
        for j in range(C // L):
            pos = base + j * L + iota16
            v = vals_m[pl.ds(voff + j * L, L)]
            acc = acc + jnp.where(pos < MAT_N[g], v, 0.0)
        plsc.addupdate_scatter(partial_v, [jnp.full((L,), g, jnp.int32)], acc)
        voff += C

    for d in big_descs:
        d.wait()
    for d in zdescs:
        d.wait()
    zdescs = [
        pltpu.async_copy(zeros_v, smap.at[pl.ds(zbase + t * ZBUF, ZBUF)], sem_z)
        for t in range(8, 16)
    ]
    for d in zdescs:
        d.wait()

    geo_acc = [jnp.zeros((L,), jnp.float32) for _ in range(3)]
    for p in range(2):
        own_blk = p * 32 + tid
        wdummy = SMAP_WD + sid * 32
        rdummy = SMAP_RD + sid * 32

        if p == 1:
            for zb in range(0, 16, 8):
                zd = [
                    pltpu.async_copy(zeros_v,
                                     smap.at[pl.ds(zbase + t * ZBUF, ZBUF)],
                                     sem_z)
                    for t in range(zb, zb + 8)
                ]
                for d in zd:
                    d.wait()

        for cc in range(GT_CHUNKS):
            stg = [
                pltpu.async_copy(gtp_r.at[pl.ds(cc * GT_CH, GT_CH)], sg_r,
                                 sem_big),
                pltpu.async_copy(gtp_c.at[pl.ds(cc * GT_CH, GT_CH)], sg_c,
                                 sem_big),
                pltpu.async_copy(gtp_ov.at[pl.ds(cc * GT_CH, GT_CH)], sg_ov,
                                 sem_big),
            ]
            for d in stg:
                d.wait()

            def _scan_pairs(i, c):
                for k in range(8):
                    o = i * 128 + k * L
                    rv = sg_r[pl.ds(o, L)]
                    cv = sg_c[pl.ds(o, L)]
                    ov = sg_ov[pl.ds(o, L)]
                    owned = jnp.right_shift(rv, 5) == own_blk
                    msk = owned & (ov > 0.1)
                    lidx = jnp.bitwise_and(rv, 511) * N_SIDE + cv
                    idx_a[i, pl.ds(k * L, L)] = jnp.where(msk, lidx, wdummy)
                return c
            lax.fori_loop(0, GT_CH // 128, _scan_pairs, 0)
            for b in range(0, GT_CH // 128, 16):
                sdescs = [
                    pltpu.async_copy(ones_v, smap.at[idx_a.at[t]], sem_s,
                                     add=True)
                    for t in range(b, b + 16)
                ]
                for d in sdescs:
                    d.wait()

        for g in range(3):
            row0 = Q_ROW0[g]
            off = Q_OFF[g]

            def _scan_q(i, c, row0=row0, off=off, n=Q_N[g]):
                for k in range(8):
                    o = off + i * 128 + k * L
                    rv = sq_r[pl.ds(o, L)]
                    cv = sq_c[pl.ds(o, L)]
                    pos = i * 128 + k * L + iota16
                    owned = jnp.right_shift(rv, 5) == own_blk
                    ok = owned & (pos < n)
                    lidx = jnp.bitwise_and(rv, 511) * N_SIDE + cv
                    idx_q[row0 + i, pl.ds(k * L, L)] = jnp.where(
                        ok, lidx, rdummy)
                return c
            lax.fori_loop(0, Q_ROWS[g], _scan_q, 0)

        for b in range(0, 40, 8):
            gdescs = [
                pltpu.async_copy(smap.at[idx_q.at[t]], gq.at[t], sem_g)
                for t in range(b, min(b + 8, 42))
            ]
            for d in gdescs:
                d.wait()
        pltpu.async_copy(smap.at[idx_q.at[40]], gq.at[40], sem_g).wait()
        pltpu.async_copy(smap.at[idx_q.at[41]], gq.at[41], sem_g).wait()

        for g in range(3):
            def _acc_q(i, a, row0=Q_ROW0[g]):
                for k in range(8):
                    v = gq[row0 + i, pl.ds(k * L, L)]
                    a = a + jnp.where(v > 0.0, 1.0, 0.0)
                return a
            geo_acc[g] = lax.fori_loop(0, Q_ROWS[g], _acc_q, geo_acc[g])

    for g in range(3):
        plsc.addupdate_scatter(partial_v, [jnp.full((L,), 6 + g, jnp.int32)],
                               geo_acc[g])

    pltpu.sync_copy(partial_v, partials_out.at[tid])


_sc_call = functools.partial(
    pl.kernel,
    out_type=jax.ShapeDtypeStruct((NC * NS, L), jnp.float32),
    mesh=plsc.VectorSubcoreMesh(core_axis_name="c", subcore_axis_name="s"),
    scratch_types=[
        pltpu.VMEM((GT_CH,), jnp.int32),
        pltpu.VMEM((GT_CH,), jnp.int32),
        pltpu.VMEM((GT_CH,), jnp.float32),
        pltpu.VMEM((Q_TOTAL,), jnp.int32),
        pltpu.VMEM((Q_TOTAL,), jnp.int32),
        pltpu.VMEM((GT_CH // 128, 128), jnp.int32),
        pltpu.VMEM((42, 128), jnp.int32),
        pltpu.VMEM((42, 128), jnp.float32),
        pltpu.VMEM((ZBUF,), jnp.float32),
        pltpu.VMEM((128,), jnp.float32),
        pltpu.VMEM((1152,), jnp.int32),
        pltpu.VMEM((1152,), jnp.int32),
        pltpu.VMEM((9, 128), jnp.int32),
        pltpu.VMEM((1152,), jnp.float32),
        pltpu.VMEM((L,), jnp.float32),
        pltpu.VMEM_SHARED((SMAP_LEN,), jnp.float32),
        pltpu.SemaphoreType.DMA,
        pltpu.SemaphoreType.DMA,
        pltpu.SemaphoreType.DMA,
        pltpu.SemaphoreType.DMA,
        pltpu.SemaphoreType.DMA,
    ],
    compiler_params=pltpu.CompilerParams(needs_layout_passes=False),
)(_sc_body)


def _pad_to(x, n, fill=0):
    return jnp.concatenate([x, jnp.full((n - x.shape[0],), fill, x.dtype)])


def kernel(gt_corr_matrix, pred_corr, pred_corr_1_2, pred_corr_1_4,
           pred_corr_0_9, pred_corr_0_95, pred_corr_1, num_corr_0_9,
           num_corr_0_95, num_corr_1, ref_points_sel_c, src_points_sel_c,
           gt_node_corr_overlaps, gt_node_corr_indices,
           ref_node_corr_indices, src_node_corr_indices,
           ref_node_corr_indices_m, src_node_corr_indices_m,
           ref_node_corr_indices_s, src_node_corr_indices_s):
    gt_flat = gt_corr_matrix.reshape(-1)

    mats = (pred_corr, pred_corr_1_2, pred_corr_1_4, pred_corr_0_9,
            pred_corr_0_95, pred_corr_1)
    mat_r = jnp.concatenate(
        [_pad_to(m[:, 0].astype(jnp.int32), p)
         for m, p in zip(mats, MAT_PADDED)])
    mat_c = jnp.concatenate(
        [_pad_to(m[:, 1].astype(jnp.int32), p)
         for m, p in zip(mats, MAT_PADDED)])

    gtp_r = _pad_to(gt_node_corr_indices[:, 0].astype(jnp.int32), GT_PAD)
    gtp_c = _pad_to(gt_node_corr_indices[:, 1].astype(jnp.int32), GT_PAD)
    gtp_ov = _pad_to(gt_node_corr_overlaps.astype(jnp.float32), GT_PAD)

    q_refs = (ref_node_corr_indices, ref_node_corr_indices_m,
              ref_node_corr_indices_s)
    q_srcs = (src_node_corr_indices, src_node_corr_indices_m,
              src_node_corr_indices_s)
    q_r = jnp.concatenate(
        [_pad_to(q.astype(jnp.int32), p) for q, p in zip(q_refs, Q_PAD)])
    q_c = jnp.concatenate(
        [_pad_to(q.astype(jnp.int32), p) for q, p in zip(q_srcs, Q_PAD)])

    partials = _sc_call(gt_flat, mat_r, mat_c, gtp_r, gtp_c, gtp_ov,
                        q_r, q_c)
    sums = partials.sum(axis=0)

    return jnp.stack([
        sums[0] / MAT_N[0], sums[1] / MAT_N[1], sums[2] / MAT_N[2],
        sums[3] / MAT_N[3], sums[4] / MAT_N[4], sums[5] / MAT_N[5],
        jnp.float32(num_corr_0_9), jnp.float32(num_corr_0_95),
        jnp.float32(num_corr_1),
        sums[6] / Q_N[0], sums[7] / Q_N[1], sums[8] / Q_N[2],
    ])


# speedup vs baseline: 1.0303x; 1.0303x over previous
import functools

import jax
import jax.numpy as jnp
from jax import lax
from jax.experimental import pallas as pl
from jax.experimental.pallas import tpu as pltpu
from jax.experimental.pallas import tpu_sc as plsc

NC = 2
NS = 16
L = 16

N_SIDE = 2048

MAT_N = (5000, 2500, 1250, 4000, 4500, 5000)
MAT_C = (256, 128, 128, 128, 256, 256)
MAT_PADDED = tuple(32 * c for c in MAT_C)
MAT_OFF = (0, 8192, 12288, 16384, 20480, 28672)

GT_N = 20000
GT_CH = 2048
GT_PAD = 20480
GT_CHUNKS = GT_PAD // GT_CH

Q_N = (3000, 1500, 750)
Q_PAD = (3072, 1536, 768)
Q_OFF = (0, 3072, 4608)
Q_TOTAL = 5376
Q_ROWS = (24, 12, 6)
Q_ROW0 = (0, 24, 36)

SMAP_BODY = 512 * 2048
SMAP_WD = SMAP_BODY
SMAP_RD = SMAP_BODY + 512
SMAP_LEN = SMAP_BODY + 1024

ZBUF = 2048


def _sc_body(gt_flat, mat_r, mat_c, gtp_r, gtp_c, gtp_ov, q_r, q_c,
             partials_out,
             sg_r, sg_c, sg_ov, sq_r, sq_c, idx_a, idx_b, idx_q, vals,
             zeros_v, ones_v, idx_m, partial_v,
             smap, sem_big, sem_z, sem_m, sem_s, sem_g):
    cid = lax.axis_index("c")
    sid = lax.axis_index("s")
    tid = cid * NS + sid
    wid = sid * NC + cid
    iota16 = lax.iota(jnp.int32, L)

    big_descs = [
        pltpu.async_copy(q_r, sq_r, sem_big),
        pltpu.async_copy(q_c, sq_c, sem_big),
    ]

    def _init_zeros(i, c):
        zeros_v[pl.ds(i * L, L)] = jnp.zeros((L,), jnp.float32)
        return c
    lax.fori_loop(0, ZBUF // L, _init_zeros, 0)
    for t in range(8):
        ones_v[pl.ds(t * L, L)] = jnp.full((L,), 1.0, jnp.float32)
    partial_v[...] = jnp.zeros((L,), jnp.float32)

    zbase = sid * 65536
    zdescs = [
        pltpu.async_copy(zeros_v, smap.at[pl.ds(zbase + t * ZBUF, ZBUF)], sem_z)
        for t in range(8)
    ]
    zdescs.append(pltpu.async_copy(zeros_v.at[pl.ds(0, 32)],
                                   smap.at[pl.ds(SMAP_RD + sid * 32, 32)],
                                   sem_z))

    MTOT = sum(MAT_C)
    stg = []
    voff = 0
    for g in range(6):
        C = MAT_C[g]
        base = wid * C
        stg.append(pltpu.async_copy(mat_r.at[pl.ds(MAT_OFF[g] + base, C)],
                                    sg_r.at[pl.ds(voff, C)], sem_m))
        stg.append(pltpu.async_copy(mat_c.at[pl.ds(MAT_OFF[g] + base, C)],
                                    sg_c.at[pl.ds(voff, C)], sem_m))
        voff += C
    for d in stg:
        d.wait()
    for j in range(MTOT // L):
        rv = sg_r[pl.ds(j * L, L)]
        cv = sg_c[pl.ds(j * L, L)]
        idx_m[j // 8, pl.ds((j % 8) * L, L)] = rv * N_SIDE + cv
    mdescs = [
        pltpu.async_copy(gt_flat.at[idx_m.at[t]],
                         vals.at[pl.ds(t * 128, 128)], sem_m)
        for t in range(MTOT // 128)
    ]
    for d in mdescs:
        d.wait()
    voff = 0
    for g in range(6):
        C = MAT_C[g]
        base = wid * C
        acc = jnp.zeros((L,), jnp.float32)
        for j in range(C // L):
            pos = base + j * L + iota16
            v = vals[pl.ds(voff + j * L, L)]
            acc = acc + jnp.where(pos < MAT_N[g], v, 0.0)
        plsc.addupdate_scatter(partial_v, [jnp.full((L,), g, jnp.int32)], acc)
        voff += C

    for d in big_descs:
        d.wait()
    for d in zdescs:
        d.wait()
    for zb in range(8, 32, 8):
        zdescs = [
            pltpu.async_copy(zeros_v, smap.at[pl.ds(zbase + t * ZBUF, ZBUF)],
                             sem_z)
            for t in range(zb, zb + 8)
        ]
        for d in zdescs:
            d.wait()

    geo_acc = [jnp.zeros((L,), jnp.float32) for _ in range(3)]
    wdummy = SMAP_WD + sid * 32
    rdummy = SMAP_RD + sid * 32
    blk0 = tid
    blk1 = 32 + tid

    for cc in range(GT_CHUNKS):
        stg = [
            pltpu.async_copy(gtp_r.at[pl.ds(cc * GT_CH, GT_CH)], sg_r,
                             sem_big),
            pltpu.async_copy(gtp_c.at[pl.ds(cc * GT_CH, GT_CH)], sg_c,
                             sem_big),
            pltpu.async_copy(gtp_ov.at[pl.ds(cc * GT_CH, GT_CH)], sg_ov,
                             sem_big),
        ]
        for d in stg:
            d.wait()

        def _scan_pairs(i, c, cc=cc):
            for k in range(8):
                o = i * 128 + k * L
                rv = sg_r[pl.ds(o, L)]
                cv = sg_c[pl.ds(o, L)]
                ov = sg_ov[pl.ds(o, L)]
                hit = ov > 0.1
                blk = jnp.right_shift(rv, 5)
                lidx = jnp.bitwise_and(rv, 511) * N_SIDE + cv
                m0 = hit & (blk == blk0)
                m1 = hit & (blk == blk1)
                idx_a[i, pl.ds(k * L, L)] = jnp.where(m0, lidx, wdummy)
                idx_b[cc * (GT_CH // 128) + i, pl.ds(k * L, L)] = jnp.where(
                    m1, lidx, wdummy)
            return c
        lax.fori_loop(0, GT_CH // 128, _scan_pairs, 0)
        for b in range(0, GT_CH // 128, 16):
            sdescs = [
                pltpu.async_copy(ones_v, smap.at[idx_a.at[t]], sem_s,
                                 add=True)
                for t in range(b, b + 16)
            ]
            for d in sdescs:
                d.wait()

    for p in range(2):
        own_blk = p * 32 + tid

        if p == 1:
            for b in range(0, GT_PAD // 128, 16):
                sdescs = [
                    pltpu.async_copy(ones_v, smap.at[idx_b.at[t]], sem_s,
                                     add=True)
                    for t in range(b, b + 16)
                ]
                for d in sdescs:
                    d.wait()

        for g in range(3):
            row0 = Q_ROW0[g]
            off = Q_OFF[g]

            def _scan_q(i, c, row0=row0, off=off, n=Q_N[g]):
                for k in range(8):
                    o = off + i * 128 + k * L
                    rv = sq_r[pl.ds(o, L)]
                    cv = sq_c[pl.ds(o, L)]
                    pos = i * 128 + k * L + iota16
                    owned = jnp.right_shift(rv, 5) == own_blk
                    ok = owned & (pos < n)
                    lidx = jnp.bitwise_and(rv, 511) * N_SIDE + cv
                    idx_q[row0 + i, pl.ds(k * L, L)] = jnp.where(
                        ok, lidx, rdummy)
                return c
            lax.fori_loop(0, Q_ROWS[g], _scan_q, 0)

        for b in range(0, 40, 8):
            gdescs = [
                pltpu.async_copy(smap.at[idx_q.at[t]],
                                 vals.at[pl.ds(t * 128, 128)], sem_g)
                for t in range(b, min(b + 8, 42))
            ]
            for d in gdescs:
                d.wait()
        pltpu.async_copy(smap.at[idx_q.at[40]],
                         vals.at[pl.ds(40 * 128, 128)], sem_g).wait()
        pltpu.async_copy(smap.at[idx_q.at[41]],
                         vals.at[pl.ds(41 * 128, 128)], sem_g).wait()

        for g in range(3):
            def _acc_q(i, a, row0=Q_ROW0[g]):
                for k in range(8):
                    v = vals[pl.ds((row0 + i) * 128 + k * L, L)]
                    a = a + jnp.where(v > 0.0, 1.0, 0.0)
                return a
            geo_acc[g] = lax.fori_loop(0, Q_ROWS[g], _acc_q, geo_acc[g])

        if p == 0:
            for zb in range(0, 32, 8):
                zd = [
                    pltpu.async_copy(zeros_v,
                                     smap.at[pl.ds(zbase + t * ZBUF, ZBUF)],
                                     sem_z)
                    for t in range(zb, zb + 8)
                ]
                for d in zd:
                    d.wait()

    for g in range(3):
        plsc.addupdate_scatter(partial_v, [jnp.full((L,), 6 + g, jnp.int32)],
                               geo_acc[g])

    pltpu.sync_copy(partial_v, partials_out.at[tid])


_sc_call = functools.partial(
    pl.kernel,
    out_type=jax.ShapeDtypeStruct((NC * NS, L), jnp.float32),
    mesh=plsc.VectorSubcoreMesh(core_axis_name="c", subcore_axis_name="s"),
    scratch_types=[
        pltpu.VMEM((GT_CH,), jnp.int32),
        pltpu.VMEM((GT_CH,), jnp.int32),
        pltpu.VMEM((GT_CH,), jnp.float32),
        pltpu.VMEM((Q_TOTAL,), jnp.int32),
        pltpu.VMEM((Q_TOTAL,), jnp.int32),
        pltpu.VMEM((GT_CH // 128, 128), jnp.int32),
        pltpu.VMEM((GT_PAD // 128, 128), jnp.int32),
        pltpu.VMEM((42, 128), jnp.int32),
        pltpu.VMEM((5376,), jnp.float32),
        pltpu.VMEM((ZBUF,), jnp.float32),
        pltpu.VMEM((128,), jnp.float32),
        pltpu.VMEM((9, 128), jnp.int32),
        pltpu.VMEM((L,), jnp.float32),
        pltpu.VMEM_SHARED((SMAP_LEN,), jnp.float32),
        pltpu.SemaphoreType.DMA,
        pltpu.SemaphoreType.DMA,
        pltpu.SemaphoreType.DMA,
        pltpu.SemaphoreType.DMA,
        pltpu.SemaphoreType.DMA,
    ],
    compiler_params=pltpu.CompilerParams(needs_layout_passes=False),
)(_sc_body)


def _pad_to(x, n, fill=0):
    return jnp.concatenate([x, jnp.full((n - x.shape[0],), fill, x.dtype)])


def kernel(gt_corr_matrix, pred_corr, pred_corr_1_2, pred_corr_1_4,
           pred_corr_0_9, pred_corr_0_95, pred_corr_1, num_corr_0_9,
           num_corr_0_95, num_corr_1, ref_points_sel_c, src_points_sel_c,
           gt_node_corr_overlaps, gt_node_corr_indices,
           ref_node_corr_indices, src_node_corr_indices,
           ref_node_corr_indices_m, src_node_corr_indices_m,
           ref_node_corr_indices_s, src_node_corr_indices_s):
    gt_flat = gt_corr_matrix.reshape(-1)

    mats = (pred_corr, pred_corr_1_2, pred_corr_1_4, pred_corr_0_9,
            pred_corr_0_95, pred_corr_1)
    mat_r = jnp.concatenate(
        [_pad_to(m[:, 0].astype(jnp.int32), p)
         for m, p in zip(mats, MAT_PADDED)])
    mat_c = jnp.concatenate(
        [_pad_to(m[:, 1].astype(jnp.int32), p)
         for m, p in zip(mats, MAT_PADDED)])

    gtp_r = _pad_to(gt_node_corr_indices[:, 0].astype(jnp.int32), GT_PAD)
    gtp_c = _pad_to(gt_node_corr_indices[:, 1].astype(jnp.int32), GT_PAD)
    gtp_ov = _pad_to(gt_node_corr_overlaps.astype(jnp.float32), GT_PAD)

    q_refs = (ref_node_corr_indices, ref_node_corr_indices_m,
              ref_node_corr_indices_s)
    q_srcs = (src_node_corr_indices, src_node_corr_indices_m,
              src_node_corr_indices_s)
    q_r = jnp.concatenate(
        [_pad_to(q.astype(jnp.int32), p) for q, p in zip(q_refs, Q_PAD)])
    q_c = jnp.concatenate(
        [_pad_to(q.astype(jnp.int32), p) for q, p in zip(q_srcs, Q_PAD)])

    partials = _sc_call(gt_flat, mat_r, mat_c, gtp_r, gtp_c, gtp_ov,
                        q_r, q_c)
    sums = partials.sum(axis=0)

    return jnp.stack([
        sums[0] / MAT_N[0], sums[1] / MAT_N[1], sums[2] / MAT_N[2],
        sums[3] / MAT_N[3], sums[4] / MAT_N[4], sums[5] / MAT_N[5],
        jnp.float32(num_corr_0_9), jnp.float32(num_corr_0_95),
        jnp.float32(num_corr_1),
        sums[6] / Q_N[0], sums[7] / Q_N[1], sums[8] / Q_N[2],
    ])
